# baseline (speedup 1.0000x reference)
import jax
import jax.numpy as jnp
from jax.experimental import pallas as pl
from jax.experimental.pallas import tpu as pltpu

_B, _D, _C, _P = 1024, 256, 8, 512
_PC = _P // _C
_LAM = 1.0


def _glmvq_body(y_ref, x_ref, pg_ref, om_ref, out_ref,
                pos_ref, neg_ref, reg_ref):
    c = pl.program_id(0)

    @pl.when(c == 0)
    def _init():
        pos_ref[...] = jnp.zeros((1, _B), jnp.float32)
        neg_ref[...] = jnp.full((1, _B), jnp.inf, jnp.float32)
        reg_ref[0, 0] = 0.0

    om_c = om_ref[0]
    omb = om_c.astype(jnp.bfloat16)
    xb = x_ref[...]
    txT = jax.lax.dot_general(omb, xb, (((1,), (1,)), ((), ())),
                              preferred_element_type=jnp.float32)
    ntx = jnp.sum(txT * txT, axis=0, keepdims=True)
    pc = pg_ref[0]
    tp = jax.lax.dot_general(pc, omb, (((1,), (1,)), ((), ())),
                             preferred_element_type=jnp.float32)
    ntp = jnp.sum(tp * tp, axis=1, keepdims=True)
    crossT = jax.lax.dot_general(tp.astype(jnp.bfloat16),
                                 txT.astype(jnp.bfloat16),
                                 (((1,), (0,)), ((), ())),
                                 preferred_element_type=jnp.float32)
    dmin = jnp.min(ntp - 2.0 * crossT, axis=0, keepdims=True) + ntx
    is_c = y_ref[...] == c
    pos_ref[...] += jnp.where(is_c, dmin, 0.0)
    neg_ref[...] = jnp.minimum(neg_ref[...], jnp.where(is_c, jnp.inf, dmin))
    reg_ref[0, 0] += jnp.sum(om_c * om_c)

    @pl.when(c == _C - 1)
    def _fini():
        pos = pos_ref[...]
        neg = neg_ref[...]
        mu = (pos - neg) / (pos + neg)
        sig = 1.0 / (1.0 + jnp.exp(-_LAM * mu))
        out_ref[0, 0] = jnp.sum(sig) / _B + 0.01 * jnp.sqrt(reg_ref[0, 0])


def kernel(x, y, prototypes, omega):
    pg = prototypes.reshape(_PC, _C, _D).transpose(1, 0, 2).astype(jnp.bfloat16)
    xb = x.astype(jnp.bfloat16)
    y_row = y.reshape(1, _B)
    out = pl.pallas_call(
        _glmvq_body,
        grid=(_C,),
        in_specs=[
            pl.BlockSpec((1, _B), lambda c: (0, 0)),
            pl.BlockSpec((_B, _D), lambda c: (0, 0)),
            pl.BlockSpec((1, _PC, _D), lambda c: (c, 0, 0)),
            pl.BlockSpec((1, _D, _D), lambda c: (c, 0, 0)),
        ],
        out_specs=pl.BlockSpec(memory_space=pltpu.SMEM),
        out_shape=jax.ShapeDtypeStruct((1, 1), jnp.float32),
        scratch_shapes=[
            pltpu.VMEM((1, _B), jnp.float32),
            pltpu.VMEM((1, _B), jnp.float32),
            pltpu.SMEM((1, 1), jnp.float32),
        ],
        compiler_params=pltpu.CompilerParams(
            dimension_semantics=("arbitrary",)),
    )(y_row, xb, pg, omega)
    return out[0, 0]


# speedup vs baseline: 1.1548x; 1.1548x over previous
import jax
import jax.numpy as jnp
from jax.experimental import pallas as pl
from jax.experimental.pallas import tpu as pltpu

_B, _D, _C, _P = 1024, 256, 8, 512
_PC = _P // _C
_LAM = 1.0


def _glmvq_body(y_ref, x_ref, pg_ref, om_ref, out_ref,
                xb_ref, pos_ref, neg_ref, reg_ref):
    c = pl.program_id(0)

    @pl.when(c == 0)
    def _init():
        xb_ref[...] = x_ref[...].astype(jnp.bfloat16)
        pos_ref[...] = jnp.zeros((1, _B), jnp.float32)
        neg_ref[...] = jnp.full((1, _B), jnp.inf, jnp.float32)
        reg_ref[0, 0] = 0.0

    om_c = om_ref[0]
    omb = om_c.astype(jnp.bfloat16)
    txT = jax.lax.dot_general(omb, xb_ref[...], (((1,), (1,)), ((), ())),
                              preferred_element_type=jnp.float32)
    ntx = jnp.sum(txT * txT, axis=0, keepdims=True)
    pc = pg_ref[...].astype(jnp.bfloat16)
    tp = jax.lax.dot_general(pc, omb, (((1,), (1,)), ((), ())),
                             preferred_element_type=jnp.float32)
    ntp = jnp.sum(tp * tp, axis=1, keepdims=True)
    crossT = jax.lax.dot_general(tp.astype(jnp.bfloat16),
                                 txT.astype(jnp.bfloat16),
                                 (((1,), (0,)), ((), ())),
                                 preferred_element_type=jnp.float32)
    dmin = jnp.min(ntp - 2.0 * crossT, axis=0, keepdims=True) + ntx
    is_c = y_ref[...] == c
    pos_ref[...] += jnp.where(is_c, dmin, 0.0)
    neg_ref[...] = jnp.minimum(neg_ref[...], jnp.where(is_c, jnp.inf, dmin))
    reg_ref[0, 0] += jnp.sum(om_c * om_c)

    @pl.when(c == _C - 1)
    def _fini():
        pos = pos_ref[...]
        neg = neg_ref[...]
        mu = (pos - neg) / (pos + neg)
        sig = 1.0 / (1.0 + jnp.exp(-_LAM * mu))
        out_ref[0, 0] = jnp.sum(sig) / _B + 0.01 * jnp.sqrt(reg_ref[0, 0])


def kernel(x, y, prototypes, omega):
    pg = prototypes.reshape(_PC, _C * _D)
    y_row = y.reshape(1, _B)
    out = pl.pallas_call(
        _glmvq_body,
        grid=(_C,),
        in_specs=[
            pl.BlockSpec((1, _B), lambda c: (0, 0)),
            pl.BlockSpec((_B, _D), lambda c: (0, 0)),
            pl.BlockSpec((_PC, _D), lambda c: (0, c)),
            pl.BlockSpec((1, _D, _D), lambda c: (c, 0, 0)),
        ],
        out_specs=pl.BlockSpec(memory_space=pltpu.SMEM),
        out_shape=jax.ShapeDtypeStruct((1, 1), jnp.float32),
        scratch_shapes=[
            pltpu.VMEM((_B, _D), jnp.bfloat16),
            pltpu.VMEM((1, _B), jnp.float32),
            pltpu.VMEM((1, _B), jnp.float32),
            pltpu.SMEM((1, 1), jnp.float32),
        ],
        compiler_params=pltpu.CompilerParams(
            dimension_semantics=("arbitrary",)),
    )(y_row, x, pg, omega)
    return out[0, 0]


# speedup vs baseline: 1.3518x; 1.1706x over previous
import jax
import jax.numpy as jnp
from jax.experimental import pallas as pl
from jax.experimental.pallas import tpu as pltpu

_B, _D, _C, _P = 1024, 256, 8, 512
_PC = _P // _C
_LAM = 1.0


def _glmvq_body(y_ref, x_ref, pg_ref, om_ref, out_ref):
    xb = x_ref[...].astype(jnp.bfloat16)
    yrow = y_ref[...]
    pos = jnp.zeros((1, _B), jnp.float32)
    neg = jnp.full((1, _B), jnp.inf, jnp.float32)
    for c in range(_C):
        omb = om_ref[c].astype(jnp.bfloat16)
        txT = jax.lax.dot_general(omb, xb, (((1,), (1,)), ((), ())),
                                  preferred_element_type=jnp.float32)
        ntx = jnp.sum(txT * txT, axis=0, keepdims=True)
        pc = pg_ref[:, c * _D:(c + 1) * _D].astype(jnp.bfloat16)
        tp = jax.lax.dot_general(pc, omb, (((1,), (1,)), ((), ())),
                                 preferred_element_type=jnp.float32)
        ntp = jnp.sum(tp * tp, axis=1, keepdims=True)
        crossT = jax.lax.dot_general(tp.astype(jnp.bfloat16),
                                     txT.astype(jnp.bfloat16),
                                     (((1,), (0,)), ((), ())),
                                     preferred_element_type=jnp.float32)
        dmin = jnp.min(ntp - 2.0 * crossT, axis=0, keepdims=True) + ntx
        is_c = yrow == c
        pos = pos + jnp.where(is_c, dmin, 0.0)
        neg = jnp.minimum(neg, jnp.where(is_c, jnp.inf, dmin))
    mu = (pos - neg) / (pos + neg)
    sig = 1.0 / (1.0 + jnp.exp(-_LAM * mu))
    om = om_ref[...]
    reg = jnp.sqrt(jnp.sum(om * om))
    out_ref[0, 0] = jnp.sum(sig) / _B + 0.01 * reg


def kernel(x, y, prototypes, omega):
    pg = prototypes.reshape(_PC, _C * _D)
    y_row = y.reshape(1, _B)
    out = pl.pallas_call(
        _glmvq_body,
        out_shape=jax.ShapeDtypeStruct((1, 1), jnp.float32),
        out_specs=pl.BlockSpec(memory_space=pltpu.SMEM),
    )(y_row, x, pg, omega)
    return out[0, 0]
